# baseline (speedup 1.0000x reference)
import functools

import jax
import jax.numpy as jnp
from jax import lax
from jax.experimental import pallas as pl
from jax.experimental.pallas import tpu as pltpu
from jax.experimental.pallas import tpu_sc as plsc

EMBED_DIM = 64
CHUNK = 128


@jax.jit
def _embed(idx3, weight):
    info = plsc.get_sparse_core_info()
    nw = info.num_cores * info.num_subcores
    n_chunks = idx3.shape[1]
    per_w = n_chunks * CHUNK
    n = nw * per_w

    mesh = plsc.VectorSubcoreMesh(core_axis_name="c", subcore_axis_name="s")

    @functools.partial(
        pl.kernel,
        mesh=mesh,
        compiler_params=pltpu.CompilerParams(use_tc_tiling_on_sc=False),
        out_type=jax.ShapeDtypeStruct((n, EMBED_DIM), jnp.float32),
        scratch_types=[
            pltpu.VMEM((n_chunks, CHUNK), jnp.int32),
            pltpu.VMEM((CHUNK, EMBED_DIM), jnp.float32),
            pltpu.SemaphoreType.DMA,
        ],
    )
    def emb(idx_hbm, table_hbm, out_hbm, idx_v, rows_v, gsem):
        wid = lax.axis_index("s") * info.num_cores + lax.axis_index("c")
        base = wid * per_w
        pltpu.sync_copy(idx_hbm.at[wid], idx_v)

        def body(j, carry):
            pltpu.async_copy(table_hbm.at[idx_v.at[j]], rows_v, gsem).wait()
            pltpu.sync_copy(rows_v, out_hbm.at[pl.ds(base + j * CHUNK, CHUNK)])
            return carry

        lax.fori_loop(0, n_chunks, body, 0)

    return emb(idx3, weight)


def kernel(input, weight):
    b, h = input.shape
    n = b * h
    info = plsc.get_sparse_core_info()
    nw = info.num_cores * info.num_subcores
    idx3 = input.reshape(nw, n // (nw * CHUNK), CHUNK).astype(jnp.int32)
    out = _embed(idx3, weight)
    return out.reshape(b, h, weight.shape[1])


# speedup vs baseline: 1.0173x; 1.0173x over previous
import functools

import jax
import jax.numpy as jnp
from jax import lax
from jax.experimental import pallas as pl
from jax.experimental.pallas import tpu as pltpu
from jax.experimental.pallas import tpu_sc as plsc

EMBED_DIM = 64
CHUNK = 128


@jax.jit
def _embed(idx3, weight):
    info = plsc.get_sparse_core_info()
    nw = info.num_cores * info.num_subcores
    n_chunks = idx3.shape[1]
    per_w = n_chunks * CHUNK
    n = nw * per_w

    mesh = plsc.VectorSubcoreMesh(core_axis_name="c", subcore_axis_name="s")

    @functools.partial(
        pl.kernel,
        mesh=mesh,
        compiler_params=pltpu.CompilerParams(use_tc_tiling_on_sc=False),
        out_type=jax.ShapeDtypeStruct((n, EMBED_DIM), jnp.float32),
        scratch_types=[
            pltpu.VMEM((n_chunks, CHUNK), jnp.int32),
            pltpu.VMEM((2, CHUNK, EMBED_DIM), jnp.float32),
            pltpu.SemaphoreType.DMA,
            pltpu.SemaphoreType.DMA,
        ],
    )
    def emb(idx_hbm, table_hbm, out_hbm, idx_v, rows_v, gsem, ssem):
        wid = lax.axis_index("s") * info.num_cores + lax.axis_index("c")
        base = wid * per_w
        pltpu.sync_copy(idx_hbm.at[wid], idx_v)

        def gather(j, b):
            return pltpu.make_async_copy(
                table_hbm.at[idx_v.at[j]], rows_v.at[b], gsem
            )

        def scatter(j, b):
            return pltpu.make_async_copy(
                rows_v.at[b], out_hbm.at[pl.ds(base + j * CHUNK, CHUNK)], ssem
            )

        gather(0, 0).start()
        gather(0, 0).wait()
        gather(1, 1).start()
        scatter(0, 0).start()

        def body(j, carry):
            b = lax.rem(j, 2)
            gather(j, b).wait()
            scatter(j - 1, 1 - b).wait()
            gather(j + 1, 1 - b).start()
            scatter(j, b).start()
            return carry

        lax.fori_loop(1, n_chunks - 1, body, 0)

        jl = n_chunks - 1
        bl = (n_chunks - 1) % 2
        gather(jl, bl).wait()
        scatter(jl - 1, 1 - bl).wait()
        scatter(jl, bl).start()
        scatter(jl, bl).wait()

    return emb(idx3, weight)


def kernel(input, weight):
    b, h = input.shape
    n = b * h
    info = plsc.get_sparse_core_info()
    nw = info.num_cores * info.num_subcores
    idx3 = input.reshape(nw, n // (nw * CHUNK), CHUNK).astype(jnp.int32)
    out = _embed(idx3, weight)
    return out.reshape(b, h, weight.shape[1])


# speedup vs baseline: 1.0381x; 1.0204x over previous
import functools

import jax
import jax.numpy as jnp
from jax import lax
from jax.experimental import pallas as pl
from jax.experimental.pallas import tpu as pltpu
from jax.experimental.pallas import tpu_sc as plsc

EMBED_DIM = 64
CHUNK = 128
K = 5


@jax.jit
def _embed(idx3, weight):
    info = plsc.get_sparse_core_info()
    nw = info.num_cores * info.num_subcores
    n_chunks = idx3.shape[1]
    per_w = n_chunks * CHUNK
    n = nw * per_w
    n_megas = n_chunks // K
    mega_rows = K * CHUNK

    mesh = plsc.VectorSubcoreMesh(core_axis_name="c", subcore_axis_name="s")

    @functools.partial(
        pl.kernel,
        mesh=mesh,
        compiler_params=pltpu.CompilerParams(use_tc_tiling_on_sc=False),
        out_type=jax.ShapeDtypeStruct((n, EMBED_DIM), jnp.float32),
        scratch_types=[
            pltpu.VMEM((n_chunks, CHUNK), jnp.int32),
            pltpu.VMEM((2, mega_rows, EMBED_DIM), jnp.float32),
            pltpu.SemaphoreType.DMA,
            pltpu.SemaphoreType.DMA,
            pltpu.SemaphoreType.DMA,
            pltpu.SemaphoreType.DMA,
        ],
    )
    def emb(idx_hbm, table_hbm, out_hbm, idx_v, rows_v, g0, g1, s0, s1):
        wid = lax.axis_index("s") * info.num_cores + lax.axis_index("c")
        base = wid * per_w
        pltpu.sync_copy(idx_hbm.at[wid], idx_v)

        gsem = (g0, g1)
        ssem = (s0, s1)

        def gath(m, k, b):
            return pltpu.make_async_copy(
                table_hbm.at[idx_v.at[m * K + k]],
                rows_v.at[b].at[pl.ds(k * CHUNK, CHUNK)],
                gsem[b],
            )

        def fire(m, b):
            for k in range(K):
                gath(m, k, b).start()

        def drain(m, b):
            for k in range(K):
                gath(m, k, b).wait()

        def scat(m, b):
            return pltpu.make_async_copy(
                rows_v.at[b],
                out_hbm.at[pl.ds(base + m * mega_rows, mega_rows)],
                ssem[b],
            )

        fire(0, 0)
        fire(1, 1)

        def body(t, carry):
            m0 = 2 * t
            m1 = m0 + 1
            drain(m0, 0)
            scat(m0, 0).start()
            drain(m1, 1)
            scat(m1, 1).start()
            scat(m0, 0).wait()
            fire(m0 + 2, 0)
            scat(m1, 1).wait()
            fire(m1 + 2, 1)
            return carry

        lax.fori_loop(0, n_megas // 2 - 1, body, 0)

        ml = n_megas - 2
        drain(ml, 0)
        scat(ml, 0).start()
        drain(ml + 1, 1)
        scat(ml + 1, 1).start()
        scat(ml, 0).wait()
        scat(ml + 1, 1).wait()

    return emb(idx3, weight)


def kernel(input, weight):
    b, h = input.shape
    n = b * h
    info = plsc.get_sparse_core_info()
    nw = info.num_cores * info.num_subcores
    idx3 = input.reshape(nw, n // (nw * CHUNK), CHUNK).astype(jnp.int32)
    out = _embed(idx3, weight)
    return out.reshape(b, h, weight.shape[1])


# speedup vs baseline: 1.0417x; 1.0035x over previous
import functools

import jax
import jax.numpy as jnp
from jax import lax
from jax.experimental import pallas as pl
from jax.experimental.pallas import tpu as pltpu
from jax.experimental.pallas import tpu_sc as plsc

EMBED_DIM = 64
CHUNK = 128
K = 5


@jax.jit
def _embed(idx3, weight):
    info = plsc.get_sparse_core_info()
    nw = info.num_cores * info.num_subcores
    n_chunks = idx3.shape[1]
    per_w = n_chunks * CHUNK
    n = nw * per_w
    n_megas = n_chunks // K
    mega_rows = K * CHUNK

    mesh = plsc.VectorSubcoreMesh(core_axis_name="c", subcore_axis_name="s")
    NB = 3

    @functools.partial(
        pl.kernel,
        mesh=mesh,
        compiler_params=pltpu.CompilerParams(use_tc_tiling_on_sc=False),
        out_type=jax.ShapeDtypeStruct((n, EMBED_DIM), jnp.float32),
        scratch_types=[
            pltpu.VMEM((n_chunks, CHUNK), jnp.int32),
            pltpu.VMEM((NB, mega_rows, EMBED_DIM), jnp.float32),
            pltpu.SemaphoreType.DMA,
            pltpu.SemaphoreType.DMA,
            pltpu.SemaphoreType.DMA,
            pltpu.SemaphoreType.DMA,
            pltpu.SemaphoreType.DMA,
            pltpu.SemaphoreType.DMA,
        ],
    )
    def emb(idx_hbm, table_hbm, out_hbm, idx_v, rows_v, g0, g1, g2, s0, s1, s2):
        wid = lax.axis_index("s") * info.num_cores + lax.axis_index("c")
        base = wid * per_w
        pltpu.sync_copy(idx_hbm.at[wid], idx_v)

        gsem = (g0, g1, g2)
        ssem = (s0, s1, s2)

        def gath(m, k, b):
            return pltpu.make_async_copy(
                table_hbm.at[idx_v.at[m * K + k]],
                rows_v.at[b].at[pl.ds(k * CHUNK, CHUNK)],
                gsem[b],
            )

        def fire(m, b):
            for k in range(K):
                gath(m, k, b).start()

        def drain(m, b):
            for k in range(K):
                gath(m, k, b).wait()

        def scat(m, b):
            return pltpu.make_async_copy(
                rows_v.at[b],
                out_hbm.at[pl.ds(base + m * mega_rows, mega_rows)],
                ssem[b],
            )

        for b in range(NB):
            fire(b, b)

        for m in range(n_megas):
            b = m % NB
            drain(m, b)
            scat(m, b).start()
            if m + NB < n_megas:
                scat(m, b).wait()
                fire(m + NB, b)

        for m in range(n_megas - NB, n_megas):
            scat(m, m % NB).wait()

    return emb(idx3, weight)


def kernel(input, weight):
    b, h = input.shape
    n = b * h
    info = plsc.get_sparse_core_info()
    nw = info.num_cores * info.num_subcores
    idx3 = input.reshape(nw, n // (nw * CHUNK), CHUNK).astype(jnp.int32)
    out = _embed(idx3, weight)
    return out.reshape(b, h, weight.shape[1])


# speedup vs baseline: 1.0427x; 1.0010x over previous
import functools

import jax
import jax.numpy as jnp
from jax import lax
from jax.experimental import pallas as pl
from jax.experimental.pallas import tpu as pltpu
from jax.experimental.pallas import tpu_sc as plsc

EMBED_DIM = 64
CHUNK = 64
K = 10


@jax.jit
def _embed(idx3, weight):
    info = plsc.get_sparse_core_info()
    nw = info.num_cores * info.num_subcores
    n_chunks = idx3.shape[1]
    per_w = n_chunks * CHUNK
    n = nw * per_w
    n_megas = n_chunks // K
    mega_rows = K * CHUNK

    mesh = plsc.VectorSubcoreMesh(core_axis_name="c", subcore_axis_name="s")
    NB = 3

    @functools.partial(
        pl.kernel,
        mesh=mesh,
        compiler_params=pltpu.CompilerParams(use_tc_tiling_on_sc=False),
        out_type=jax.ShapeDtypeStruct((n, EMBED_DIM), jnp.float32),
        scratch_types=[
            pltpu.VMEM((n_chunks, CHUNK), jnp.int32),
            pltpu.VMEM((NB, mega_rows, EMBED_DIM), jnp.float32),
            pltpu.SemaphoreType.DMA,
            pltpu.SemaphoreType.DMA,
            pltpu.SemaphoreType.DMA,
            pltpu.SemaphoreType.DMA,
            pltpu.SemaphoreType.DMA,
            pltpu.SemaphoreType.DMA,
        ],
    )
    def emb(idx_hbm, table_hbm, out_hbm, idx_v, rows_v, g0, g1, g2, s0, s1, s2):
        wid = lax.axis_index("s") * info.num_cores + lax.axis_index("c")
        base = wid * per_w
        pltpu.sync_copy(idx_hbm.at[wid], idx_v)

        gsem = (g0, g1, g2)
        ssem = (s0, s1, s2)

        def gath(m, k, b):
            return pltpu.make_async_copy(
                table_hbm.at[idx_v.at[m * K + k]],
                rows_v.at[b].at[pl.ds(k * CHUNK, CHUNK)],
                gsem[b],
            )

        def fire(m, b):
            for k in range(K):
                gath(m, k, b).start()

        def drain(m, b):
            for k in range(K):
                gath(m, k, b).wait()

        def scat(m, b):
            return pltpu.make_async_copy(
                rows_v.at[b],
                out_hbm.at[pl.ds(base + m * mega_rows, mega_rows)],
                ssem[b],
            )

        for b in range(NB):
            fire(b, b)

        for m in range(n_megas):
            b = m % NB
            drain(m, b)
            scat(m, b).start()
            if m + NB < n_megas:
                scat(m, b).wait()
                fire(m + NB, b)

        for m in range(n_megas - NB, n_megas):
            scat(m, m % NB).wait()

    return emb(idx3, weight)


def kernel(input, weight):
    b, h = input.shape
    n = b * h
    info = plsc.get_sparse_core_info()
    nw = info.num_cores * info.num_subcores
    idx3 = input.reshape(nw, n // (nw * CHUNK), CHUNK).astype(jnp.int32)
    out = _embed(idx3, weight)
    return out.reshape(b, h, weight.shape[1])


# speedup vs baseline: 1.0435x; 1.0008x over previous
import functools

import jax
import jax.numpy as jnp
from jax import lax
from jax.experimental import pallas as pl
from jax.experimental.pallas import tpu as pltpu
from jax.experimental.pallas import tpu_sc as plsc

EMBED_DIM = 64
CHUNK = 128
K = 5


@jax.jit
def _embed(idx3, weight):
    info = plsc.get_sparse_core_info()
    nw = info.num_cores * info.num_subcores
    n_chunks = idx3.shape[1]
    per_w = n_chunks * CHUNK
    n = nw * per_w
    n_megas = n_chunks // K
    mega_rows = K * CHUNK

    mesh = plsc.VectorSubcoreMesh(core_axis_name="c", subcore_axis_name="s")
    NB = 3

    @functools.partial(
        pl.kernel,
        mesh=mesh,
        compiler_params=pltpu.CompilerParams(use_tc_tiling_on_sc=False),
        out_type=jax.ShapeDtypeStruct((n, EMBED_DIM), jnp.float32),
        scratch_types=[
            pltpu.VMEM((n_chunks, CHUNK), jnp.int32),
            pltpu.VMEM((NB, mega_rows, EMBED_DIM), jnp.float32),
            pltpu.SemaphoreType.DMA,
            pltpu.SemaphoreType.DMA,
            pltpu.SemaphoreType.DMA,
            pltpu.SemaphoreType.DMA,
            pltpu.SemaphoreType.DMA,
            pltpu.SemaphoreType.DMA,
        ],
    )
    def emb(idx_hbm, table_hbm, out_hbm, idx_v, rows_v, g0, g1, g2, s0, s1, s2):
        wid = lax.axis_index("s") * info.num_cores + lax.axis_index("c")
        base = wid * per_w
        pltpu.sync_copy(idx_hbm.at[wid], idx_v)

        gsem = (g0, g1, g2)
        ssem = (s0, s1, s2)

        def gath(m, k, b):
            return pltpu.make_async_copy(
                table_hbm.at[idx_v.at[m * K + k]],
                rows_v.at[b].at[pl.ds(k * CHUNK, CHUNK)],
                gsem[b],
            )

        def fire(m, b):
            for k in range(K):
                gath(m, k, b).start()

        def drain(m, b):
            for k in range(K):
                gath(m, k, b).wait()

        def scat(m, b):
            return pltpu.make_async_copy(
                rows_v.at[b],
                out_hbm.at[pl.ds(base + m * mega_rows, mega_rows)],
                ssem[b],
            )

        for b in range(NB):
            fire(b, b)

        for m in range(n_megas):
            b = m % NB
            drain(m, b)
            scat(m, b).start()
            if m + NB < n_megas:
                scat(m, b).wait()
                fire(m + NB, b)

        for m in range(n_megas - NB, n_megas):
            scat(m, m % NB).wait()

    return emb(idx3, weight)


def kernel(input, weight):
    b, h = input.shape
    n = b * h
    info = plsc.get_sparse_core_info()
    nw = info.num_cores * info.num_subcores
    idx3 = input.reshape(nw, n // (nw * CHUNK), CHUNK).astype(jnp.int32)
    out = _embed(idx3, weight)
    return out.reshape(b, h, weight.shape[1])
